# baseline (device time: 1485537 ns/iter reference)
import jax
import jax.numpy as jnp
from jax import lax
from jax.experimental import pallas as pl
from jax.experimental.pallas import tpu as pltpu

N_DEV = 32


def kernel(x, w_mat, scale_x, scale_w):
    m, k_per = x.shape
    n = w_mat.shape[1]
    m_per = m // N_DEV
    s2 = (scale_x * scale_w).reshape(1, 1)

    def body(x_ref, w_ref, s_ref, out_ref, comm_ref, wbf_ref,
             send_sems, recv_sems, credit_sem):
        d = lax.axis_index("i")
        left = lax.rem(d - 1 + N_DEV, N_DEV)
        right = lax.rem(d + 1, N_DEV)

        barrier = pltpu.get_barrier_semaphore()
        pl.semaphore_signal(barrier, inc=1, device_id=(left,),
                            device_id_type=pl.DeviceIdType.MESH)
        pl.semaphore_signal(barrier, inc=1, device_id=(right,),
                            device_id_type=pl.DeviceIdType.MESH)
        pl.semaphore_wait(barrier, 2)

        wbf_ref[...] = w_ref[...].astype(jnp.bfloat16)

        def partial_for(c):
            xb = x_ref[pl.ds(c * m_per, m_per), :].astype(jnp.bfloat16)
            return lax.dot_general(
                xb, wbf_ref[...],
                dimension_numbers=(((1,), (0,)), ((), ())),
                preferred_element_type=jnp.float32)

        comm_ref[0] = partial_for(lax.rem(d - 1 + N_DEV, N_DEV))

        def hop(h, carry):
            c = lax.rem(d - h - 2 + 2 * N_DEV, N_DEV)

            def one(send_slot, recv_slot):
                @pl.when(h >= 2)
                def _():
                    pl.semaphore_wait(credit_sem, 1)

                rdma = pltpu.make_async_remote_copy(
                    src_ref=comm_ref.at[send_slot],
                    dst_ref=comm_ref.at[recv_slot],
                    send_sem=send_sems.at[send_slot],
                    recv_sem=recv_sems.at[recv_slot],
                    device_id=(right,),
                    device_id_type=pl.DeviceIdType.MESH)
                rdma.start()
                rdma.wait()

                comm_ref[recv_slot] = comm_ref[recv_slot] + partial_for(c)

                @pl.when(h <= N_DEV - 4)
                def _():
                    pl.semaphore_signal(
                        credit_sem, inc=1, device_id=(left,),
                        device_id_type=pl.DeviceIdType.MESH)

            @pl.when(lax.rem(h, 2) == 0)
            def _():
                one(0, 1)

            @pl.when(lax.rem(h, 2) == 1)
            def _():
                one(1, 0)

            return carry

        lax.fori_loop(0, N_DEV - 1, hop, 0)

        out_ref[...] = jnp.maximum(comm_ref[1] * s_ref[0, 0], 0.0)

    return pl.pallas_call(
        body,
        out_shape=jax.ShapeDtypeStruct((m_per, n), jnp.float32),
        in_specs=[
            pl.BlockSpec(memory_space=pltpu.VMEM),
            pl.BlockSpec(memory_space=pltpu.VMEM),
            pl.BlockSpec(memory_space=pltpu.SMEM),
        ],
        out_specs=pl.BlockSpec(memory_space=pltpu.VMEM),
        scratch_shapes=[
            pltpu.VMEM((2, m_per, n), jnp.float32),
            pltpu.VMEM((k_per, n), jnp.bfloat16),
            pltpu.SemaphoreType.DMA((2,)),
            pltpu.SemaphoreType.DMA((2,)),
            pltpu.SemaphoreType.REGULAR,
        ],
        compiler_params=pltpu.CompilerParams(collective_id=0),
    )(x, w_mat, s2)


# device time: 781941 ns/iter; 1.8998x vs baseline; 1.8998x over previous
import jax
import jax.numpy as jnp
from jax import lax
from jax.experimental import pallas as pl
from jax.experimental.pallas import tpu as pltpu

N_DEV = 32


def kernel(x, w_mat, scale_x, scale_w):
    m, k_per = x.shape
    n = w_mat.shape[1]
    nh = n // 2
    m_per = m // N_DEV
    s2 = (scale_x * scale_w).reshape(1, 1)

    def body(x_ref, w_ref, s_ref, out_ref, comm_r, comm_l, wbf_ref,
             send_sems_r, recv_sems_r, send_sems_l, recv_sems_l,
             credit_r, credit_l):
        d = lax.axis_index("i")
        left = lax.rem(d - 1 + N_DEV, N_DEV)
        right = lax.rem(d + 1, N_DEV)

        barrier = pltpu.get_barrier_semaphore()
        pl.semaphore_signal(barrier, inc=1, device_id=(left,),
                            device_id_type=pl.DeviceIdType.MESH)
        pl.semaphore_signal(barrier, inc=1, device_id=(right,),
                            device_id_type=pl.DeviceIdType.MESH)
        pl.semaphore_wait(barrier, 2)

        wbf_ref[...] = w_ref[...].astype(jnp.bfloat16)

        def partial_lo(c):
            xb = x_ref[pl.ds(c * m_per, m_per), :].astype(jnp.bfloat16)
            return lax.dot_general(
                xb, wbf_ref[:, :nh],
                dimension_numbers=(((1,), (0,)), ((), ())),
                preferred_element_type=jnp.float32)

        def partial_hi(c):
            xb = x_ref[pl.ds(c * m_per, m_per), :].astype(jnp.bfloat16)
            return lax.dot_general(
                xb, wbf_ref[:, nh:],
                dimension_numbers=(((1,), (0,)), ((), ())),
                preferred_element_type=jnp.float32)

        comm_r[0] = partial_lo(lax.rem(d - 1 + N_DEV, N_DEV)).astype(jnp.bfloat16)
        comm_l[0] = partial_hi(lax.rem(d + 1, N_DEV)).astype(jnp.bfloat16)

        def hop(h, carry):
            c_r = lax.rem(d - h - 2 + 2 * N_DEV, N_DEV)
            c_l = lax.rem(d + h + 2, N_DEV)

            def one(send_slot, recv_slot):
                @pl.when(h >= 1)
                def _():
                    pl.semaphore_wait(credit_r, 1)
                    pl.semaphore_wait(credit_l, 1)

                rdma_r = pltpu.make_async_remote_copy(
                    src_ref=comm_r.at[send_slot],
                    dst_ref=comm_r.at[recv_slot],
                    send_sem=send_sems_r.at[send_slot],
                    recv_sem=recv_sems_r.at[recv_slot],
                    device_id=(right,),
                    device_id_type=pl.DeviceIdType.MESH)
                rdma_l = pltpu.make_async_remote_copy(
                    src_ref=comm_l.at[send_slot],
                    dst_ref=comm_l.at[recv_slot],
                    send_sem=send_sems_l.at[send_slot],
                    recv_sem=recv_sems_l.at[recv_slot],
                    device_id=(left,),
                    device_id_type=pl.DeviceIdType.MESH)
                rdma_r.start()
                rdma_l.start()
                rdma_r.wait()
                rdma_l.wait()

                @pl.when(h <= N_DEV - 3)
                def _():
                    pl.semaphore_signal(
                        credit_r, inc=1, device_id=(left,),
                        device_id_type=pl.DeviceIdType.MESH)
                    pl.semaphore_signal(
                        credit_l, inc=1, device_id=(right,),
                        device_id_type=pl.DeviceIdType.MESH)

                comm_r[recv_slot] = (
                    comm_r[recv_slot].astype(jnp.float32) + partial_lo(c_r)
                ).astype(jnp.bfloat16)
                comm_l[recv_slot] = (
                    comm_l[recv_slot].astype(jnp.float32) + partial_hi(c_l)
                ).astype(jnp.bfloat16)

            @pl.when(lax.rem(h, 2) == 0)
            def _():
                one(0, 1)

            @pl.when(lax.rem(h, 2) == 1)
            def _():
                one(1, 0)

            return carry

        lax.fori_loop(0, N_DEV - 1, hop, 0)

        s = s_ref[0, 0]
        out_ref[:, :nh] = jnp.maximum(
            comm_r[1].astype(jnp.float32) * s, 0.0)
        out_ref[:, nh:] = jnp.maximum(
            comm_l[1].astype(jnp.float32) * s, 0.0)

    return pl.pallas_call(
        body,
        out_shape=jax.ShapeDtypeStruct((m_per, n), jnp.float32),
        in_specs=[
            pl.BlockSpec(memory_space=pltpu.VMEM),
            pl.BlockSpec(memory_space=pltpu.VMEM),
            pl.BlockSpec(memory_space=pltpu.SMEM),
        ],
        out_specs=pl.BlockSpec(memory_space=pltpu.VMEM),
        scratch_shapes=[
            pltpu.VMEM((2, m_per, nh), jnp.bfloat16),
            pltpu.VMEM((2, m_per, nh), jnp.bfloat16),
            pltpu.VMEM((k_per, n), jnp.bfloat16),
            pltpu.SemaphoreType.DMA((2,)),
            pltpu.SemaphoreType.DMA((2,)),
            pltpu.SemaphoreType.DMA((2,)),
            pltpu.SemaphoreType.DMA((2,)),
            pltpu.SemaphoreType.REGULAR,
            pltpu.SemaphoreType.REGULAR,
        ],
        compiler_params=pltpu.CompilerParams(collective_id=0),
    )(x, w_mat, s2)


# device time: 770344 ns/iter; 1.9284x vs baseline; 1.0151x over previous
import jax
import jax.numpy as jnp
from jax import lax
from jax.experimental import pallas as pl
from jax.experimental.pallas import tpu as pltpu

N_DEV = 32
N_SLOTS = 3


def kernel(x, w_mat, scale_x, scale_w):
    m, k_per = x.shape
    n = w_mat.shape[1]
    nh = n // 2
    m_per = m // N_DEV
    s2 = (scale_x * scale_w).reshape(1, 1)

    def body(x_ref, w_ref, s_ref, out_ref, comm_r, comm_l, wbf_ref,
             stage_r, stage_l,
             send_sems_r, recv_sems_r, send_sems_l, recv_sems_l,
             credit_r, credit_l):
        d = lax.axis_index("i")
        left = lax.rem(d - 1 + N_DEV, N_DEV)
        right = lax.rem(d + 1, N_DEV)

        barrier = pltpu.get_barrier_semaphore()
        pl.semaphore_signal(barrier, inc=1, device_id=(left,),
                            device_id_type=pl.DeviceIdType.MESH)
        pl.semaphore_signal(barrier, inc=1, device_id=(right,),
                            device_id_type=pl.DeviceIdType.MESH)
        pl.semaphore_wait(barrier, 2)

        wbf_ref[...] = w_ref[...].astype(jnp.bfloat16)

        def partial_lo(c):
            xb = x_ref[pl.ds(c * m_per, m_per), :].astype(jnp.bfloat16)
            return lax.dot_general(
                xb, wbf_ref[:, :nh],
                dimension_numbers=(((1,), (0,)), ((), ())),
                preferred_element_type=jnp.float32)

        def partial_hi(c):
            xb = x_ref[pl.ds(c * m_per, m_per), :].astype(jnp.bfloat16)
            return lax.dot_general(
                xb, wbf_ref[:, nh:],
                dimension_numbers=(((1,), (0,)), ((), ())),
                preferred_element_type=jnp.float32)

        comm_r[0] = partial_lo(lax.rem(d - 1 + N_DEV, N_DEV)).astype(jnp.bfloat16)
        comm_l[0] = partial_hi(lax.rem(d + 1, N_DEV)).astype(jnp.bfloat16)

        def make_pair(slot):
            send_slot, recv_slot = slot, (slot + 1) % N_SLOTS
            rdma_r = pltpu.make_async_remote_copy(
                src_ref=comm_r.at[send_slot],
                dst_ref=comm_r.at[recv_slot],
                send_sem=send_sems_r.at[send_slot],
                recv_sem=recv_sems_r.at[recv_slot],
                device_id=(right,),
                device_id_type=pl.DeviceIdType.MESH)
            rdma_l = pltpu.make_async_remote_copy(
                src_ref=comm_l.at[send_slot],
                dst_ref=comm_l.at[recv_slot],
                send_sem=send_sems_l.at[send_slot],
                recv_sem=recv_sems_l.at[recv_slot],
                device_id=(left,),
                device_id_type=pl.DeviceIdType.MESH)
            return rdma_r, rdma_l

        def hop(h, carry):
            c_r = lax.rem(d - h - 2 + 2 * N_DEV, N_DEV)
            c_l = lax.rem(d + h + 2, N_DEV)
            recv_slot = lax.rem(h + 1, N_SLOTS)

            @pl.when(h >= 2)
            def _():
                pl.semaphore_wait(credit_r, 1)
                pl.semaphore_wait(credit_l, 1)

            for slot in range(N_SLOTS):
                @pl.when(lax.rem(h, N_SLOTS) == slot)
                def _(slot=slot):
                    rdma_r, rdma_l = make_pair(slot)
                    rdma_r.start()
                    rdma_l.start()

            stage_r[...] = partial_lo(c_r)
            stage_l[...] = partial_hi(c_l)

            for slot in range(N_SLOTS):
                @pl.when(lax.rem(h, N_SLOTS) == slot)
                def _(slot=slot):
                    rdma_r, rdma_l = make_pair(slot)
                    rdma_r.wait()
                    rdma_l.wait()

            @pl.when(h <= N_DEV - 4)
            def _():
                pl.semaphore_signal(
                    credit_r, inc=1, device_id=(left,),
                    device_id_type=pl.DeviceIdType.MESH)
                pl.semaphore_signal(
                    credit_l, inc=1, device_id=(right,),
                    device_id_type=pl.DeviceIdType.MESH)

            comm_r[recv_slot] = (
                comm_r[recv_slot].astype(jnp.float32) + stage_r[...]
            ).astype(jnp.bfloat16)
            comm_l[recv_slot] = (
                comm_l[recv_slot].astype(jnp.float32) + stage_l[...]
            ).astype(jnp.bfloat16)

            return carry

        lax.fori_loop(0, N_DEV - 1, hop, 0)

        s = s_ref[0, 0]
        out_ref[:, :nh] = jnp.maximum(
            comm_r[1].astype(jnp.float32) * s, 0.0)
        out_ref[:, nh:] = jnp.maximum(
            comm_l[1].astype(jnp.float32) * s, 0.0)

    return pl.pallas_call(
        body,
        out_shape=jax.ShapeDtypeStruct((m_per, n), jnp.float32),
        in_specs=[
            pl.BlockSpec(memory_space=pltpu.VMEM),
            pl.BlockSpec(memory_space=pltpu.VMEM),
            pl.BlockSpec(memory_space=pltpu.SMEM),
        ],
        out_specs=pl.BlockSpec(memory_space=pltpu.VMEM),
        scratch_shapes=[
            pltpu.VMEM((N_SLOTS, m_per, nh), jnp.bfloat16),
            pltpu.VMEM((N_SLOTS, m_per, nh), jnp.bfloat16),
            pltpu.VMEM((k_per, n), jnp.bfloat16),
            pltpu.VMEM((m_per, nh), jnp.float32),
            pltpu.VMEM((m_per, nh), jnp.float32),
            pltpu.SemaphoreType.DMA((N_SLOTS,)),
            pltpu.SemaphoreType.DMA((N_SLOTS,)),
            pltpu.SemaphoreType.DMA((N_SLOTS,)),
            pltpu.SemaphoreType.DMA((N_SLOTS,)),
            pltpu.SemaphoreType.REGULAR,
            pltpu.SemaphoreType.REGULAR,
        ],
        compiler_params=pltpu.CompilerParams(collective_id=0),
    )(x, w_mat, s2)
